# Initial kernel scaffold; baseline (speedup 1.0000x reference)
import functools

import jax
import jax.numpy as jnp
from jax.experimental import pallas as pl

NUM_GRAPHS = 512
NPG = 116
NN = NUM_GRAPHS * NPG
NE = NN * 10
INV6 = 1.0 / 6.0


def _aug_body(newx_ref, w1_ref, b1_ref, w2_ref, b2_ref, f1w_ref, f1b_ref,
              f2w_ref, f2b_ref, z2_ref):
    xg = newx_ref[0]
    f32 = jnp.float32
    sim = jax.lax.dot_general(xg, xg, (((1,), (1,)), ((), ())),
                              preferred_element_type=f32)
    row = jax.lax.broadcasted_iota(jnp.int32, (NPG, NPG), 0)
    col = jax.lax.broadcasted_iota(jnp.int32, (NPG, NPG), 1)
    adj = (row == col).astype(f32)
    simw = sim
    for _ in range(5):
        m = jnp.max(simw, axis=1, keepdims=True)
        sel = simw == m
        first = sel & (jnp.cumsum(sel.astype(jnp.int32), axis=1) == 1)
        adj = adj + first.astype(f32)
        simw = jnp.where(first, -jnp.inf, simw)
    h1 = jnp.dot(xg, w1_ref[...], preferred_element_type=f32)
    hh1 = jnp.maximum(jnp.dot(adj, h1, preferred_element_type=f32) * INV6
                      + b1_ref[...], 0.0)
    h2 = jnp.dot(hh1, w2_ref[...], preferred_element_type=f32)
    hh2 = jnp.maximum(jnp.dot(adj, h2, preferred_element_type=f32) * INV6
                      + b2_ref[...], 0.0)
    p = jnp.sum(hh2, axis=0, keepdims=True) * (1.0 / NPG)
    tt = jnp.maximum(jnp.dot(p, f1w_ref[...], preferred_element_type=f32)
                     + f1b_ref[...], 0.0)
    z2 = jnp.maximum(jnp.dot(tt, f2w_ref[...], preferred_element_type=f32)
                     + f2b_ref[...], 0.0)
    z2_ref[...] = z2


def _aug_pallas(new_x3, W1, b1, W2, b2, fc1W, fc1b, fc2W, fc2b):
    full = lambda shape: pl.BlockSpec(shape, lambda i: (0,) * len(shape))
    return pl.pallas_call(
        _aug_body,
        grid=(NUM_GRAPHS,),
        in_specs=[
            pl.BlockSpec((1, NPG, NPG), lambda i: (i, 0, 0)),
            full((116, 128)), full((1, 128)),
            full((128, 64)), full((1, 64)),
            full((64, 128)), full((1, 128)),
            full((128, 128)), full((1, 128)),
        ],
        out_specs=pl.BlockSpec((1, 128), lambda i: (i, 0)),
        out_shape=jax.ShapeDtypeStruct((NUM_GRAPHS, 128), jnp.float32),
    )(new_x3, W1, b1.reshape(1, -1), W2, b2.reshape(1, -1),
      fc1W, fc1b.reshape(1, -1), fc2W, fc2b.reshape(1, -1))


def kernel(x, edge_index, batch, W1, b1, W2, b2, fc1W, fc1b, fc2W, fc2b,
           m1W, m1b, m2W, m2b, m3W, m3b, Wg1, bg1, Wg2, bg2):
    src = edge_index[0]
    dst = edge_index[1]
    deg = jax.ops.segment_sum(jnp.ones((NE,), jnp.float32), dst,
                              num_segments=NN) + 1.0
    dinv = deg ** -0.5

    Wc1 = jnp.concatenate([W1, Wg1], axis=1)
    bc1 = jnp.concatenate([b1, bg1])
    hs1 = (x @ Wc1) * dinv[:, None]
    acc1 = jax.ops.segment_sum(hs1[src], dst, num_segments=NN)
    out1 = dinv[:, None] * (acc1 + hs1) + bc1
    h = jax.nn.relu(out1[:, :128])
    g = jax.nn.relu(out1[:, 128:])

    bc2 = jnp.concatenate([b2, bg2])
    hs2 = jnp.concatenate([h @ W2, g @ Wg2], axis=1) * dinv[:, None]
    acc2 = jax.ops.segment_sum(hs2[src], dst, num_segments=NN)
    out2 = dinv[:, None] * (acc2 + hs2) + bc2
    h2 = jax.nn.relu(out2[:, :64])
    new_x = out2[:, 64:]

    z1b = h2.reshape(NUM_GRAPHS, NPG, 64).mean(axis=1)
    t = jax.nn.relu(z1b @ fc1W + fc1b)
    z1_after = jax.nn.relu(t @ fc2W + fc2b)
    logits = jax.nn.relu(jax.nn.relu(z1b @ m1W + m1b) @ m2W + m2b) @ m3W + m3b
    output = jax.nn.softmax(logits, axis=1)

    new_x3 = new_x.reshape(NUM_GRAPHS, NPG, NPG)
    z2_after = _aug_pallas(new_x3, W1, b1, W2, b2, fc1W, fc1b, fc2W, fc2b)
    return (z1_after, z2_after, output, x, new_x)


# speedup vs baseline: 4.4420x; 4.4420x over previous
import functools

import jax
import jax.numpy as jnp
from jax.experimental import pallas as pl

NUM_GRAPHS = 512
NPG = 116
NN = NUM_GRAPHS * NPG
NE = NN * 10
INV6 = 1.0 / 6.0


def _aug_body(newx_ref, w1_ref, b1_ref, w2_ref, b2_ref, f1w_ref, f1b_ref,
              f2w_ref, f2b_ref, z2_ref):
    xg = newx_ref[0]
    f32 = jnp.float32
    sim = jax.lax.dot_general(xg, xg, (((1,), (1,)), ((), ())),
                              preferred_element_type=f32)
    row = jax.lax.broadcasted_iota(jnp.int32, (NPG, NPG), 0)
    col = jax.lax.broadcasted_iota(jnp.int32, (NPG, NPG), 1)
    adj = (row == col).astype(f32)
    simw = sim
    for _ in range(5):
        m = jnp.max(simw, axis=1, keepdims=True)
        sel = simw == m
        jmin = jnp.min(jnp.where(sel, col, NPG), axis=1, keepdims=True)
        first = col == jmin
        adj = adj + first.astype(f32)
        simw = jnp.where(first, -jnp.inf, simw)
    h1 = jnp.dot(xg, w1_ref[...], preferred_element_type=f32)
    hh1 = jnp.maximum(jnp.dot(adj, h1, preferred_element_type=f32) * INV6
                      + b1_ref[...], 0.0)
    h2 = jnp.dot(hh1, w2_ref[...], preferred_element_type=f32)
    hh2 = jnp.maximum(jnp.dot(adj, h2, preferred_element_type=f32) * INV6
                      + b2_ref[...], 0.0)
    p = jnp.sum(hh2, axis=0, keepdims=True) * (1.0 / NPG)
    tt = jnp.maximum(jnp.dot(p, f1w_ref[...], preferred_element_type=f32)
                     + f1b_ref[...], 0.0)
    z2 = jnp.maximum(jnp.dot(tt, f2w_ref[...], preferred_element_type=f32)
                     + f2b_ref[...], 0.0)
    z2_ref[0] = z2


def _aug_pallas(new_x3, W1, b1, W2, b2, fc1W, fc1b, fc2W, fc2b):
    full = lambda shape: pl.BlockSpec(shape, lambda i: (0,) * len(shape))
    return pl.pallas_call(
        _aug_body,
        grid=(NUM_GRAPHS,),
        in_specs=[
            pl.BlockSpec((1, NPG, NPG), lambda i: (i, 0, 0)),
            full((116, 128)), full((1, 128)),
            full((128, 64)), full((1, 64)),
            full((64, 128)), full((1, 128)),
            full((128, 128)), full((1, 128)),
        ],
        out_specs=pl.BlockSpec((1, 1, 128), lambda i: (i, 0, 0)),
        out_shape=jax.ShapeDtypeStruct((NUM_GRAPHS, 1, 128), jnp.float32),
    )(new_x3, W1, b1.reshape(1, -1), W2, b2.reshape(1, -1),
      fc1W, fc1b.reshape(1, -1), fc2W, fc2b.reshape(1, -1)).reshape(NUM_GRAPHS, 128)


def kernel(x, edge_index, batch, W1, b1, W2, b2, fc1W, fc1b, fc2W, fc2b,
           m1W, m1b, m2W, m2b, m3W, m3b, Wg1, bg1, Wg2, bg2):
    src = edge_index[0]
    dst = edge_index[1]
    deg = jax.ops.segment_sum(jnp.ones((NE,), jnp.float32), dst,
                              num_segments=NN) + 1.0
    dinv = deg ** -0.5

    Wc1 = jnp.concatenate([W1, Wg1], axis=1)
    bc1 = jnp.concatenate([b1, bg1])
    hs1 = (x @ Wc1) * dinv[:, None]
    acc1 = jax.ops.segment_sum(hs1[src], dst, num_segments=NN)
    out1 = dinv[:, None] * (acc1 + hs1) + bc1
    h = jax.nn.relu(out1[:, :128])
    g = jax.nn.relu(out1[:, 128:])

    bc2 = jnp.concatenate([b2, bg2])
    hs2 = jnp.concatenate([h @ W2, g @ Wg2], axis=1) * dinv[:, None]
    acc2 = jax.ops.segment_sum(hs2[src], dst, num_segments=NN)
    out2 = dinv[:, None] * (acc2 + hs2) + bc2
    h2 = jax.nn.relu(out2[:, :64])
    new_x = out2[:, 64:]

    z1b = h2.reshape(NUM_GRAPHS, NPG, 64).mean(axis=1)
    t = jax.nn.relu(z1b @ fc1W + fc1b)
    z1_after = jax.nn.relu(t @ fc2W + fc2b)
    logits = jax.nn.relu(jax.nn.relu(z1b @ m1W + m1b) @ m2W + m2b) @ m3W + m3b
    output = jax.nn.softmax(logits, axis=1)

    new_x3 = new_x.reshape(NUM_GRAPHS, NPG, NPG)
    z2_after = _aug_pallas(new_x3, W1, b1, W2, b2, fc1W, fc1b, fc2W, fc2b)
    return (z1_after, z2_after, output, x, new_x)


# speedup vs baseline: 14.6712x; 3.3028x over previous
import functools

import jax
import jax.numpy as jnp
from jax import lax
from jax.experimental import pallas as pl
from jax.experimental.pallas import tpu as pltpu
from jax.experimental.pallas import tpu_sc as plsc

NUM_GRAPHS = 512
NPG = 116
NN = NUM_GRAPHS * NPG
NE = NN * 10
INV6 = 1.0 / 6.0

NCORE = 2
NSUB = 16
FC = 32
NCHUNK = 6
CPC = NCHUNK // NCORE
ROWS_PER_TILE = NN // NSUB
NE_PAD = 622592
EDGE_PAD = NE_PAD - NE
ACC_ROWS = NN + 16
DEG_WIN = NE_PAD // (NCORE * NSUB) // 128
IDXW = 512
GW = 128
NWIN_G = NE_PAD // IDXW


def _sc_deg(dst2d):
    mesh = plsc.VectorSubcoreMesh(core_axis_name="c", subcore_axis_name="s")

    @functools.partial(
        pl.kernel, mesh=mesh,
        out_type=jax.ShapeDtypeStruct((NCORE * NN,), jnp.float32),
        scratch_types=[
            pltpu.VMEM((DEG_WIN, 128), jnp.int32),
            pltpu.VMEM((128,), jnp.float32),
            pltpu.VMEM((ROWS_PER_TILE,), jnp.float32),
            pltpu.VMEM_SHARED((ACC_ROWS,), jnp.float32),
        ],
    )
    def k(dst_hbm, out_hbm, idx_v, ones_v, zrow_v, acc):
        c = lax.axis_index("c")
        s = lax.axis_index("s")
        wid = c * NSUB + s

        def fill(i, _):
            ones_v[pl.ds(i * 16, 16)] = jnp.full((16,), 1.0, jnp.float32)
            return 0
        lax.fori_loop(0, 8, fill, 0)

        def zfill(i, _):
            zrow_v[pl.ds(i * 16, 16)] = jnp.zeros((16,), jnp.float32)
            return 0
        lax.fori_loop(0, ROWS_PER_TILE // 16, zfill, 0)

        pltpu.sync_copy(dst_hbm.at[pl.ds(wid * DEG_WIN, DEG_WIN)], idx_v)
        pltpu.sync_copy(zrow_v, acc.at[pl.ds(s * ROWS_PER_TILE, ROWS_PER_TILE)])
        @pl.when(s == 0)
        def _():
            pltpu.sync_copy(zrow_v.at[pl.ds(0, 16)], acc.at[pl.ds(NN, 16)])
        plsc.subcore_barrier()

        def win(w, _):
            pltpu.sync_copy(ones_v, acc.at[idx_v.at[w]], add=True)
            return 0
        lax.fori_loop(0, DEG_WIN, win, 0)
        plsc.subcore_barrier()
        pltpu.sync_copy(acc.at[pl.ds(s * ROWS_PER_TILE, ROWS_PER_TILE)],
                        out_hbm.at[pl.ds(c * NN + s * ROWS_PER_TILE,
                                         ROWS_PER_TILE)])

    return k(dst2d)


def _sc_segsum(hs6, eidx1d):
    mesh = plsc.VectorSubcoreMesh(core_axis_name="c", subcore_axis_name="s")

    @functools.partial(
        pl.kernel, mesh=mesh,
        compiler_params=pltpu.CompilerParams(use_tc_tiling_on_sc=False),
        out_type=jax.ShapeDtypeStruct((NCHUNK * NN, FC), jnp.float32),
        scratch_types=[
            pltpu.VMEM((2 * IDXW,), jnp.int32),
            pltpu.VMEM((2 * IDXW,), jnp.int32),
            pltpu.VMEM((GW, FC), jnp.float32),
            pltpu.VMEM((GW, FC), jnp.float32),
            pltpu.VMEM_SHARED((ACC_ROWS, FC), jnp.float32),
            pltpu.SemaphoreType.DMA,
            pltpu.SemaphoreType.DMA,
            pltpu.SemaphoreType.DMA,
            pltpu.SemaphoreType.DMA,
            pltpu.SemaphoreType.DMA,
            pltpu.SemaphoreType.DMA,
        ],
    )
    def k(hs_hbm, eidx_hbm, out_hbm, eidx_a, eidx_b, rows_a, rows_b,
          acc, isa, isb, gsa, gsb, ssa, ssb):
        c = lax.axis_index("c")
        s = lax.axis_index("s")
        nwin = NE_PAD // NSUB // IDXW
        nsub = IDXW // GW
        eidx = (eidx_a, eidx_b)
        rows = (rows_a, rows_b)
        isem = (isa, isb)
        gsem = (gsa, gsb)
        ssem = (ssa, ssb)

        def idx_fetch(kk, w, b):
            off = (CPC * c + kk) * (2 * NE_PAD) + (s * nwin + w) * (2 * IDXW)
            return pltpu.async_copy(eidx_hbm.at[pl.ds(off, 2 * IDXW)],
                                    eidx[b], isem[b])

        for kk in range(CPC):
            base = (CPC * c + kk) * NN
            pltpu.sync_copy(
                hs_hbm.at[pl.ds(base + s * ROWS_PER_TILE, ROWS_PER_TILE)],
                acc.at[pl.ds(s * ROWS_PER_TILE, ROWS_PER_TILE)])
            idx_fetch(kk, 0, 0).wait()
            plsc.subcore_barrier()

            def win(w, _):
                b = lax.rem(w, 2)

                def sub_pipe(buf):
                    src_w = eidx[buf].at[pl.ds(0, IDXW)]
                    dst_w = eidx[buf].at[pl.ds(IDXW, IDXW)]
                    gd = [None, None]
                    sd = [None, None]
                    gd[0] = pltpu.async_copy(
                        hs_hbm.at[src_w.at[pl.ds(0, GW)]], rows[0], gsem[0])
                    for j in range(nsub):
                        jb = j % 2
                        gd[jb].wait()
                        if sd[jb] is not None:
                            sd[jb].wait()
                        sd[jb] = pltpu.async_copy(
                            rows[jb], acc.at[dst_w.at[pl.ds(j * GW, GW)]],
                            ssem[jb], add=True)
                        if j + 1 < nsub:
                            nb = (j + 1) % 2
                            if sd[nb] is not None:
                                sd[nb].wait()
                                sd[nb] = None
                            gd[nb] = pltpu.async_copy(
                                hs_hbm.at[src_w.at[pl.ds((j + 1) * GW, GW)]],
                                rows[nb], gsem[nb])
                    for d in sd:
                        if d is not None:
                            d.wait()

                def one_phase(buf):
                    @pl.when(b == buf)
                    def _():
                        @pl.when(w + 1 < nwin)
                        def _():
                            idx_fetch(kk, w + 1, 1 - buf)
                        sub_pipe(buf)
                        @pl.when(w + 1 < nwin)
                        def _():
                            pltpu.make_async_copy(
                                eidx_hbm.at[pl.ds(0, 2 * IDXW)],
                                eidx[1 - buf], isem[1 - buf]).wait()

                one_phase(0)
                one_phase(1)
                return 0
            lax.fori_loop(0, nwin, win, 0)
            plsc.subcore_barrier()
            pltpu.sync_copy(
                acc.at[pl.ds(s * ROWS_PER_TILE, ROWS_PER_TILE)],
                out_hbm.at[pl.ds(base + s * ROWS_PER_TILE, ROWS_PER_TILE)])
            if kk < CPC - 1:
                plsc.subcore_barrier()

    return k(hs6, eidx1d)


AUG_G = 16


def _aug_body(newx_ref, w1_ref, b1_ref, w2_ref, b2_ref, f1w_ref, f1b_ref,
              f2w_ref, f2b_ref, z2_ref):
    f32 = jnp.float32
    col = jax.lax.broadcasted_iota(jnp.int32, (AUG_G, NPG, NPG), 2)
    row = jax.lax.broadcasted_iota(jnp.int32, (AUG_G, NPG, NPG), 1)
    xga = newx_ref[...]
    sim = jax.lax.dot_general(xga, xga, (((2,), (2,)), ((0,), (0,))),
                              preferred_element_type=f32)
    adj = (row == col).astype(f32)
    simw = sim
    for _ in range(5):
        m = jnp.max(simw, axis=2, keepdims=True)
        sel = simw == m
        jmin = jnp.min(jnp.where(sel, col, NPG), axis=2, keepdims=True)
        first = col == jmin
        adj = adj + first.astype(f32)
        simw = jnp.where(first, -jnp.inf, simw)
    h1 = jnp.dot(xga, w1_ref[...], preferred_element_type=f32)
    hh1 = jnp.maximum(
        jax.lax.dot_general(adj, h1, (((2,), (1,)), ((0,), (0,))),
                            preferred_element_type=f32) * INV6
        + b1_ref[...], 0.0)
    h2 = jnp.dot(hh1, w2_ref[...], preferred_element_type=f32)
    hh2 = jnp.maximum(
        jax.lax.dot_general(adj, h2, (((2,), (1,)), ((0,), (0,))),
                            preferred_element_type=f32) * INV6
        + b2_ref[...], 0.0)
    p = jnp.sum(hh2, axis=1, keepdims=True) * (1.0 / NPG)
    tt = jnp.maximum(jnp.dot(p, f1w_ref[...], preferred_element_type=f32)
                     + f1b_ref[...], 0.0)
    z2 = jnp.maximum(jnp.dot(tt, f2w_ref[...], preferred_element_type=f32)
                     + f2b_ref[...], 0.0)
    z2_ref[...] = z2


def _aug_pallas(new_x3, W1, b1, W2, b2, fc1W, fc1b, fc2W, fc2b):
    full = lambda shape: pl.BlockSpec(shape, lambda i: (0,) * len(shape))
    return pl.pallas_call(
        _aug_body,
        grid=(NUM_GRAPHS // AUG_G,),
        in_specs=[
            pl.BlockSpec((AUG_G, NPG, NPG), lambda i: (i, 0, 0)),
            full((116, 128)), full((1, 128)),
            full((128, 64)), full((1, 64)),
            full((64, 128)), full((1, 128)),
            full((128, 128)), full((1, 128)),
        ],
        out_specs=pl.BlockSpec((AUG_G, 1, 128), lambda i: (i, 0, 0)),
        out_shape=jax.ShapeDtypeStruct((NUM_GRAPHS, 1, 128), jnp.float32),
    )(new_x3, W1, b1.reshape(1, -1), W2, b2.reshape(1, -1),
      fc1W, fc1b.reshape(1, -1), fc2W, fc2b.reshape(1, -1)).reshape(NUM_GRAPHS, 128)


def kernel(x, edge_index, batch, W1, b1, W2, b2, fc1W, fc1b, fc2W, fc2b,
           m1W, m1b, m2W, m2b, m3W, m3b, Wg1, bg1, Wg2, bg2):
    src = edge_index[0]
    dst = edge_index[1]
    padi = jnp.arange(EDGE_PAD, dtype=jnp.int32) % 16
    src1d = jnp.concatenate([src, padi])
    dst1d = jnp.concatenate([dst, NN + padi])
    dst2d = dst1d.reshape(NE_PAD // 128, 128)
    src_off = src1d[None, :] + (jnp.arange(NCHUNK, dtype=jnp.int32) * NN)[:, None]
    eidx1d = jnp.stack(
        [src_off.reshape(NCHUNK, NWIN_G, IDXW),
         jnp.broadcast_to(dst1d, (NCHUNK, NE_PAD)).reshape(NCHUNK, NWIN_G, IDXW)],
        axis=2).reshape(-1)

    degp = _sc_deg(dst2d)
    deg = degp[:NN] + degp[NN:] + 1.0
    dinv = deg ** -0.5

    def to_chunks(hsc):
        return hsc.reshape(NN, NCHUNK, FC).transpose(1, 0, 2).reshape(
            NCHUNK * NN, FC)

    def from_chunks(acc):
        return acc.reshape(NCHUNK, NN, FC).transpose(1, 0, 2).reshape(
            NN, NCHUNK * FC)

    Wc1 = jnp.concatenate([W1, Wg1], axis=1)
    bc1 = jnp.concatenate([b1, bg1])
    hs1 = (x @ Wc1) * dinv[:, None]
    acc1 = from_chunks(_sc_segsum(to_chunks(hs1), eidx1d))
    out1 = dinv[:, None] * acc1 + bc1
    h = jax.nn.relu(out1[:, :128])
    g = jax.nn.relu(out1[:, 128:])

    bc2 = jnp.concatenate([b2, bg2])
    hs2 = jnp.concatenate(
        [h @ W2, g @ Wg2, jnp.zeros((NN, 12), jnp.float32)],
        axis=1) * dinv[:, None]
    acc2 = from_chunks(_sc_segsum(to_chunks(hs2), eidx1d))[:, :180]
    out2 = dinv[:, None] * acc2 + bc2
    h2 = jax.nn.relu(out2[:, :64])
    new_x = out2[:, 64:]

    z1b = h2.reshape(NUM_GRAPHS, NPG, 64).mean(axis=1)
    t = jax.nn.relu(z1b @ fc1W + fc1b)
    z1_after = jax.nn.relu(t @ fc2W + fc2b)
    logits = jax.nn.relu(jax.nn.relu(z1b @ m1W + m1b) @ m2W + m2b) @ m3W + m3b
    output = jax.nn.softmax(logits, axis=1)

    new_x3 = new_x.reshape(NUM_GRAPHS, NPG, NPG)
    z2_after = _aug_pallas(new_x3, W1, b1, W2, b2, fc1W, fc1b, fc2W, fc2b)
    return (z1_after, z2_after, output, x, new_x)
